# baseline (device time: 59388 ns/iter reference)
import functools

import jax
import jax.numpy as jnp
from jax import lax
from jax.experimental import pallas as pl
from jax.experimental.pallas import tpu as pltpu

N_DEV = 8
N_ROUNDS = 3
N_LAYERS = 3
B = 64
D = 1024
H = 2048


def kernel(x, Win0, Wout0, Win1, Wout1, Win2, Wout2):
    def body(
        x_ref,
        win0_ref,
        wout0_ref,
        win1_ref,
        wout1_ref,
        win2_ref,
        wout2_ref,
        out_ref,
        win_stage,
        wout_stage,
        acc_ref,
        send_buf,
        recv_buf,
        load_sems,
        send_sems,
        recv_sems,
    ):
        my = lax.axis_index("i")

        barrier = pltpu.get_barrier_semaphore()
        for r in range(N_ROUNDS):
            pl.semaphore_signal(
                barrier,
                inc=1,
                device_id=(my ^ (1 << r),),
                device_id_type=pl.DeviceIdType.MESH,
            )
        pl.semaphore_wait(barrier, N_ROUNDS)

        wins = [win0_ref, win1_ref, win2_ref]
        wouts = [wout0_ref, wout1_ref, wout2_ref]

        x_bf = x_ref[:, :].astype(jnp.bfloat16)
        for l in range(N_LAYERS):
            cp_in = pltpu.make_async_copy(wins[l], win_stage, load_sems.at[0])
            cp_out = pltpu.make_async_copy(wouts[l], wout_stage, load_sems.at[1])
            cp_in.start()
            cp_out.start()
            cp_in.wait()
            h = jnp.maximum(
                jnp.dot(
                    x_bf,
                    win_stage[:, :].astype(jnp.bfloat16),
                    preferred_element_type=jnp.float32,
                ),
                0.0,
            ).astype(jnp.bfloat16)
            cp_out.wait()
            acc_ref[:, :] = jnp.dot(
                h,
                wout_stage[:, :].astype(jnp.bfloat16),
                preferred_element_type=jnp.float32,
            )

            for r in range(N_ROUNDS):
                idx = l * N_ROUNDS + r
                partner = my ^ (1 << r)
                send_buf[idx, :, :] = acc_ref[:, :].astype(jnp.bfloat16)
                rdma = pltpu.make_async_remote_copy(
                    src_ref=send_buf.at[idx],
                    dst_ref=recv_buf.at[idx],
                    send_sem=send_sems.at[idx],
                    recv_sem=recv_sems.at[idx],
                    device_id=(partner,),
                    device_id_type=pl.DeviceIdType.MESH,
                )
                rdma.start()
                rdma.wait()
                acc_ref[:, :] = acc_ref[:, :] + recv_buf[idx, :, :].astype(
                    jnp.float32
                )

            x_bf = acc_ref[:, :].astype(jnp.bfloat16)

        rows = B // N_DEV
        out_ref[:, :] = acc_ref[pl.ds(my * rows, rows), :]

        @functools.partial(pl.run_scoped, exit_sem=pltpu.SemaphoreType.REGULAR)
        def _(exit_sem):
            for r in range(N_ROUNDS):
                pl.semaphore_signal(
                    exit_sem,
                    inc=1,
                    device_id=(my ^ (1 << r),),
                    device_id_type=pl.DeviceIdType.MESH,
                )
            pl.semaphore_wait(exit_sem, N_ROUNDS)

    hbm = pl.BlockSpec(memory_space=pltpu.MemorySpace.HBM)
    vmem = pl.BlockSpec(memory_space=pltpu.VMEM)
    n_ex = N_LAYERS * N_ROUNDS
    return pl.pallas_call(
        body,
        out_shape=jax.ShapeDtypeStruct((B // N_DEV, D), jnp.float32),
        in_specs=[vmem, hbm, hbm, hbm, hbm, hbm, hbm],
        out_specs=vmem,
        scratch_shapes=[
            pltpu.VMEM((D, H), jnp.float32),
            pltpu.VMEM((H, D), jnp.float32),
            pltpu.VMEM((B, D), jnp.float32),
            pltpu.VMEM((n_ex, B, D), jnp.bfloat16),
            pltpu.VMEM((n_ex, B, D), jnp.bfloat16),
            pltpu.SemaphoreType.DMA((2,)),
            pltpu.SemaphoreType.DMA((n_ex,)),
            pltpu.SemaphoreType.DMA((n_ex,)),
        ],
        compiler_params=pltpu.CompilerParams(collective_id=0),
    )(x, Win0, Wout0, Win1, Wout1, Win2, Wout2)


# device time: 46279 ns/iter; 1.2833x vs baseline; 1.2833x over previous
import functools

import jax
import jax.numpy as jnp
from jax import lax
from jax.experimental import pallas as pl
from jax.experimental.pallas import tpu as pltpu

N_DEV = 8
N_ROUNDS = 3
N_LAYERS = 3
B = 64
D = 1024
H = 2048
MASKS = (1, 3, 4)


def kernel(x, Win0, Wout0, Win1, Wout1, Win2, Wout2):
    def body(
        x_ref,
        win0_ref,
        wout0_ref,
        win1_ref,
        wout1_ref,
        win2_ref,
        wout2_ref,
        out_ref,
        win_stage,
        wout_stage,
        acc_ref,
        send_buf,
        recv_buf,
        load_sems,
        send_sems,
        recv_sems,
    ):
        my = lax.axis_index("i")

        barrier = pltpu.get_barrier_semaphore()
        for m in MASKS:
            pl.semaphore_signal(
                barrier,
                inc=1,
                device_id=(my ^ m,),
                device_id_type=pl.DeviceIdType.MESH,
            )
        pl.semaphore_wait(barrier, N_ROUNDS)

        wins = [win0_ref, win1_ref, win2_ref]
        wouts = [wout0_ref, wout1_ref, wout2_ref]

        def stage(l, slot):
            cp_in = pltpu.make_async_copy(
                wins[l], win_stage.at[slot], load_sems.at[slot, 0]
            )
            cp_out = pltpu.make_async_copy(
                wouts[l], wout_stage.at[slot], load_sems.at[slot, 1]
            )
            cp_in.start()
            cp_out.start()
            return cp_in, cp_out

        pending = stage(0, 0)
        x_bf = x_ref[:, :].astype(jnp.bfloat16)
        for l in range(N_LAYERS):
            slot = l % 2
            cp_in, cp_out = pending
            cp_in.wait()
            h = jnp.maximum(
                jnp.dot(
                    x_bf,
                    win_stage[slot, :, :].astype(jnp.bfloat16),
                    preferred_element_type=jnp.float32,
                ),
                0.0,
            ).astype(jnp.bfloat16)
            cp_out.wait()
            acc_ref[:, :] = jnp.dot(
                h,
                wout_stage[slot, :, :].astype(jnp.bfloat16),
                preferred_element_type=jnp.float32,
            )
            if l + 1 < N_LAYERS:
                pending = stage(l + 1, 1 - slot)

            for r in range(N_ROUNDS):
                idx = l * N_ROUNDS + r
                partner = my ^ MASKS[r]
                send_buf[idx, :, :] = acc_ref[:, :].astype(jnp.bfloat16)
                rdma = pltpu.make_async_remote_copy(
                    src_ref=send_buf.at[idx],
                    dst_ref=recv_buf.at[idx],
                    send_sem=send_sems.at[idx],
                    recv_sem=recv_sems.at[idx],
                    device_id=(partner,),
                    device_id_type=pl.DeviceIdType.MESH,
                )
                rdma.start()
                rdma.wait()
                acc_ref[:, :] = acc_ref[:, :] + recv_buf[idx, :, :].astype(
                    jnp.float32
                )

            x_bf = acc_ref[:, :].astype(jnp.bfloat16)

        rows = B // N_DEV
        out_ref[:, :] = acc_ref[pl.ds(my * rows, rows), :]

        @functools.partial(pl.run_scoped, exit_sem=pltpu.SemaphoreType.REGULAR)
        def _(exit_sem):
            for m in MASKS:
                pl.semaphore_signal(
                    exit_sem,
                    inc=1,
                    device_id=(my ^ m,),
                    device_id_type=pl.DeviceIdType.MESH,
                )
            pl.semaphore_wait(exit_sem, N_ROUNDS)

    hbm = pl.BlockSpec(memory_space=pltpu.MemorySpace.HBM)
    vmem = pl.BlockSpec(memory_space=pltpu.VMEM)
    n_ex = N_LAYERS * N_ROUNDS
    return pl.pallas_call(
        body,
        out_shape=jax.ShapeDtypeStruct((B // N_DEV, D), jnp.float32),
        in_specs=[vmem, hbm, hbm, hbm, hbm, hbm, hbm],
        out_specs=vmem,
        scratch_shapes=[
            pltpu.VMEM((2, D, H), jnp.float32),
            pltpu.VMEM((2, H, D), jnp.float32),
            pltpu.VMEM((B, D), jnp.float32),
            pltpu.VMEM((n_ex, B, D), jnp.bfloat16),
            pltpu.VMEM((n_ex, B, D), jnp.bfloat16),
            pltpu.SemaphoreType.DMA((2, 2)),
            pltpu.SemaphoreType.DMA((n_ex,)),
            pltpu.SemaphoreType.DMA((n_ex,)),
        ],
        compiler_params=pltpu.CompilerParams(
            collective_id=0, vmem_limit_bytes=56 * 1024 * 1024
        ),
    )(x, Win0, Wout0, Win1, Wout1, Win2, Wout2)


# device time: 42085 ns/iter; 1.4111x vs baseline; 1.0997x over previous
import functools

import jax
import jax.numpy as jnp
from jax import lax
from jax.experimental import pallas as pl
from jax.experimental.pallas import tpu as pltpu

N_DEV = 8
N_ROUNDS = 3
N_LAYERS = 3
B = 64
D = 1024
H = 2048
MASKS = (1, 3, 4)


def kernel(x, Win0, Wout0, Win1, Wout1, Win2, Wout2):
    def body(
        x_ref,
        win0_ref,
        wout0_ref,
        win1_ref,
        wout1_ref,
        win2_ref,
        wout2_ref,
        out_ref,
        win_stage,
        wout_stage,
        acc_ref,
        send_buf,
        recv_buf,
        rs_send0,
        rs_send1,
        rs_send2,
        rs_recv0,
        rs_recv1,
        rs_recv2,
        load_sems,
        send_sems,
        recv_sems,
    ):
        my = lax.axis_index("i")

        wins = [win0_ref, win1_ref, win2_ref]
        wouts = [wout0_ref, wout1_ref, wout2_ref]

        def stage(l, slot):
            cp_in = pltpu.make_async_copy(
                wins[l], win_stage.at[slot], load_sems.at[slot, 0]
            )
            cp_out = pltpu.make_async_copy(
                wouts[l], wout_stage.at[slot], load_sems.at[slot, 1]
            )
            cp_in.start()
            cp_out.start()
            return cp_in, cp_out

        pending = stage(0, 0)

        barrier = pltpu.get_barrier_semaphore()
        for m in MASKS:
            pl.semaphore_signal(
                barrier,
                inc=1,
                device_id=(my ^ m,),
                device_id_type=pl.DeviceIdType.MESH,
            )
        pl.semaphore_wait(barrier, N_ROUNDS)

        x_bf = x_ref[:, :].astype(jnp.bfloat16)
        for l in range(N_LAYERS):
            slot = l % 2
            cp_in, cp_out = pending
            cp_in.wait()
            h = jnp.maximum(
                jnp.dot(
                    x_bf,
                    win_stage[slot, :, :].astype(jnp.bfloat16),
                    preferred_element_type=jnp.float32,
                ),
                0.0,
            ).astype(jnp.bfloat16)
            cp_out.wait()
            acc_ref[:, :] = jnp.dot(
                h,
                wout_stage[slot, :, :].astype(jnp.bfloat16),
                preferred_element_type=jnp.float32,
            )
            if l + 1 < N_LAYERS:
                pending = stage(l + 1, 1 - slot)
                for r in range(N_ROUNDS):
                    idx = l * N_ROUNDS + r
                    partner = my ^ MASKS[r]
                    send_buf[idx, :, :] = acc_ref[:, :].astype(jnp.bfloat16)
                    rdma = pltpu.make_async_remote_copy(
                        src_ref=send_buf.at[idx],
                        dst_ref=recv_buf.at[idx],
                        send_sem=send_sems.at[idx],
                        recv_sem=recv_sems.at[idx],
                        device_id=(partner,),
                        device_id_type=pl.DeviceIdType.MESH,
                    )
                    rdma.start()
                    rdma.wait()
                    acc_ref[:, :] = acc_ref[:, :] + recv_buf[
                        idx, :, :
                    ].astype(jnp.float32)
                x_bf = acc_ref[:, :].astype(jnp.bfloat16)

        rs_bufs = [(rs_send0, rs_recv0), (rs_send1, rs_recv1), (rs_send2, rs_recv2)]
        rs_rounds = [
            (4, 32 * (my // 4), 32 * ((my // 4) ^ 1), 32),
            (3, 16 * (my // 2), 16 * ((my ^ 3) // 2), 16),
            (1, 8 * my, 8 * (my ^ 1), 8),
        ]
        for r, (m, keep_off, send_off, nrows) in enumerate(rs_rounds):
            idx = 2 * N_ROUNDS + r
            partner = my ^ m
            sbuf, rbuf = rs_bufs[r]
            sbuf[:, :] = acc_ref[pl.ds(send_off, nrows), :].astype(jnp.bfloat16)
            rdma = pltpu.make_async_remote_copy(
                src_ref=sbuf,
                dst_ref=rbuf,
                send_sem=send_sems.at[idx],
                recv_sem=recv_sems.at[idx],
                device_id=(partner,),
                device_id_type=pl.DeviceIdType.MESH,
            )
            rdma.start()
            rdma.wait()
            acc_ref[pl.ds(keep_off, nrows), :] = acc_ref[
                pl.ds(keep_off, nrows), :
            ] + rbuf[:, :].astype(jnp.float32)

        rows = B // N_DEV
        out_ref[:, :] = acc_ref[pl.ds(my * rows, rows), :]

        @functools.partial(pl.run_scoped, exit_sem=pltpu.SemaphoreType.REGULAR)
        def _(exit_sem):
            for m in MASKS:
                pl.semaphore_signal(
                    exit_sem,
                    inc=1,
                    device_id=(my ^ m,),
                    device_id_type=pl.DeviceIdType.MESH,
                )
            pl.semaphore_wait(exit_sem, N_ROUNDS)

    hbm = pl.BlockSpec(memory_space=pltpu.MemorySpace.HBM)
    vmem = pl.BlockSpec(memory_space=pltpu.VMEM)
    n_ex = N_LAYERS * N_ROUNDS
    return pl.pallas_call(
        body,
        out_shape=jax.ShapeDtypeStruct((B // N_DEV, D), jnp.float32),
        in_specs=[vmem, hbm, hbm, hbm, hbm, hbm, hbm],
        out_specs=vmem,
        scratch_shapes=[
            pltpu.VMEM((2, D, H), jnp.float32),
            pltpu.VMEM((2, H, D), jnp.float32),
            pltpu.VMEM((B, D), jnp.float32),
            pltpu.VMEM((6, B, D), jnp.bfloat16),
            pltpu.VMEM((6, B, D), jnp.bfloat16),
            pltpu.VMEM((B // 2, D), jnp.bfloat16),
            pltpu.VMEM((B // 4, D), jnp.bfloat16),
            pltpu.VMEM((B // 8, D), jnp.bfloat16),
            pltpu.VMEM((B // 2, D), jnp.bfloat16),
            pltpu.VMEM((B // 4, D), jnp.bfloat16),
            pltpu.VMEM((B // 8, D), jnp.bfloat16),
            pltpu.SemaphoreType.DMA((2, 2)),
            pltpu.SemaphoreType.DMA((n_ex,)),
            pltpu.SemaphoreType.DMA((n_ex,)),
        ],
        compiler_params=pltpu.CompilerParams(
            collective_id=0, vmem_limit_bytes=56 * 1024 * 1024
        ),
    )(x, Win0, Wout0, Win1, Wout1, Win2, Wout2)
